# Initial kernel scaffold; baseline (speedup 1.0000x reference)
import functools
import jax
import jax.numpy as jnp
from jax import lax
from jax.experimental import pallas as pl
from jax.experimental.pallas import tpu as pltpu
from jax.experimental.pallas import tpu_sc as plsc

N = 10000
E = 320000
D = 128
C = 10
G = 64

NC = 2
NS = 16
NW = NC * NS
EPW = E // NW
CH = 80
NCHUNK = EPW // CH
RPT = N // NS
DEGW = 16

_mesh = plsc.VectorSubcoreMesh(core_axis_name="c", subcore_axis_name="s")


def _sc_agg_body(with_deg, *refs):
    if with_deg:
        (h_hbm, src_hbm, dst_hbm, zrow_hbm, zdeg_hbm,
         out_hbm, deg_hbm, acc, dacc, src_v, dst_v, rows_v, ones_v, sem) = refs
    else:
        (h_hbm, src_hbm, dst_hbm, zrow_hbm,
         out_hbm, acc, src_v, dst_v, rows_v, sem) = refs
    cid = lax.axis_index("c")
    sid = lax.axis_index("s")
    wid = sid * NC + cid

    r0 = sid * RPT
    pltpu.sync_copy(zrow_hbm.at[pl.ds(r0, RPT)], acc.at[pl.ds(r0, RPT)])
    if with_deg:
        pltpu.sync_copy(zdeg_hbm.at[pl.ds(r0, RPT)], dacc.at[pl.ds(r0, RPT)])

        def fill_ones(i, carry):
            ones_v[i] = jnp.ones((16,), jnp.float32)
            return carry
        lax.fori_loop(0, CH, fill_ones, 0)
    plsc.subcore_barrier()

    def chunk(i, carry):
        base = wid * EPW + i * CH
        pltpu.sync_copy(src_hbm.at[pl.ds(base, CH)], src_v)
        pltpu.sync_copy(dst_hbm.at[pl.ds(base, CH)], dst_v)
        pltpu.async_copy(h_hbm.at[src_v], rows_v, sem).wait()
        pltpu.sync_copy(rows_v, acc.at[dst_v], add=True)
        if with_deg:
            pltpu.sync_copy(ones_v, dacc.at[dst_v], add=True)
        return carry
    lax.fori_loop(0, NCHUNK, chunk, 0)

    plsc.subcore_barrier()
    pltpu.sync_copy(acc.at[pl.ds(r0, RPT)], out_hbm.at[cid, pl.ds(r0, RPT)])
    if with_deg:
        pltpu.sync_copy(dacc.at[pl.ds(r0, RPT)],
                        deg_hbm.at[cid, pl.ds(r0, RPT)])


_agg_deg = functools.partial(
    pl.kernel,
    out_type=(jax.ShapeDtypeStruct((NC, N, D), jnp.float32),
              jax.ShapeDtypeStruct((NC, N, DEGW), jnp.float32)),
    mesh=_mesh,
    scratch_types=[
        pltpu.VMEM_SHARED((N, D), jnp.float32),
        pltpu.VMEM_SHARED((N, DEGW), jnp.float32),
        pltpu.VMEM((CH,), jnp.int32),
        pltpu.VMEM((CH,), jnp.int32),
        pltpu.VMEM((CH, D), jnp.float32),
        pltpu.VMEM((CH, DEGW), jnp.float32),
        pltpu.SemaphoreType.DMA,
    ],
)(functools.partial(_sc_agg_body, True))

_agg_only = functools.partial(
    pl.kernel,
    out_type=jax.ShapeDtypeStruct((NC, N, D), jnp.float32),
    mesh=_mesh,
    scratch_types=[
        pltpu.VMEM_SHARED((N, D), jnp.float32),
        pltpu.VMEM((CH,), jnp.int32),
        pltpu.VMEM((CH,), jnp.int32),
        pltpu.VMEM((CH, D), jnp.float32),
        pltpu.SemaphoreType.DMA,
    ],
)(functools.partial(_sc_agg_body, False))


BN = 1000
NB = N // BN


def _embed_body(x_ref, w_ref, b_ref, o_ref):
    o_ref[...] = jnp.dot(x_ref[...], w_ref[...],
                         preferred_element_type=jnp.float32) + b_ref[...]


_embed = pl.pallas_call(
    _embed_body,
    grid=(NB,),
    in_specs=[
        pl.BlockSpec((BN, D), lambda i: (i, 0)),
        pl.BlockSpec((D, D), lambda i: (0, 0)),
        pl.BlockSpec((1, D), lambda i: (0, 0)),
    ],
    out_specs=pl.BlockSpec((BN, D), lambda i: (i, 0)),
    out_shape=jax.ShapeDtypeStruct((N, D), jnp.float32),
)


def _layer_body(a0_ref, a1_ref, d0_ref, d1_ref, w_ref, b_ref, o_ref):
    deg = jnp.maximum(d0_ref[:, :1] + d1_ref[:, :1], 1.0)
    h = (a0_ref[...] + a1_ref[...]) / deg
    z = jnp.dot(h, w_ref[...], preferred_element_type=jnp.float32) + b_ref[...]
    o_ref[...] = jnp.maximum(z, 0.0)


_layer = pl.pallas_call(
    _layer_body,
    grid=(NB,),
    in_specs=[
        pl.BlockSpec((BN, D), lambda i: (i, 0)),
        pl.BlockSpec((BN, D), lambda i: (i, 0)),
        pl.BlockSpec((BN, DEGW), lambda i: (i, 0)),
        pl.BlockSpec((BN, DEGW), lambda i: (i, 0)),
        pl.BlockSpec((D, D), lambda i: (0, 0)),
        pl.BlockSpec((1, D), lambda i: (0, 0)),
    ],
    out_specs=pl.BlockSpec((BN, D), lambda i: (i, 0)),
    out_shape=jax.ShapeDtypeStruct((N, D), jnp.float32),
)


def _head_body(a0_ref, a1_ref, d0_ref, d1_ref, gid_ref, w_ref, b_ref,
               wo_ref, bo_ref, o_ref, pooled_acc, cnt_acc):
    i = pl.program_id(0)

    @pl.when(i == 0)
    def _():
        pooled_acc[...] = jnp.zeros_like(pooled_acc)
        cnt_acc[...] = jnp.zeros_like(cnt_acc)

    deg = jnp.maximum(d0_ref[:, :1] + d1_ref[:, :1], 1.0)
    h = (a0_ref[...] + a1_ref[...]) / deg
    z = jnp.dot(h, w_ref[...], preferred_element_type=jnp.float32) + b_ref[...]
    h2 = jnp.maximum(z, 0.0)

    gid = gid_ref[...]
    gcol = lax.broadcasted_iota(jnp.int32, (BN, G), 1)
    onehot = (gid == gcol).astype(jnp.float32)
    dn = (((0,), (0,)), ((), ()))
    pooled_acc[...] += lax.dot_general(onehot, h2, dn,
                                       preferred_element_type=jnp.float32)
    cnt_acc[...] += lax.dot_general(onehot, jnp.ones((BN, D), jnp.float32),
                                    dn, preferred_element_type=jnp.float32)

    @pl.when(i == NB - 1)
    def _():
        pooled = pooled_acc[...] / jnp.maximum(cnt_acc[...], 1.0)
        logits = jnp.dot(pooled, wo_ref[...],
                         preferred_element_type=jnp.float32) + bo_ref[...]
        logits = jnp.where(logits >= 0, logits, 0.01 * logits)
        m = jnp.max(logits, axis=-1, keepdims=True)
        e = jnp.exp(logits - m)
        o_ref[...] = e / jnp.sum(e, axis=-1, keepdims=True)


_head = pl.pallas_call(
    _head_body,
    grid=(NB,),
    in_specs=[
        pl.BlockSpec((BN, D), lambda i: (i, 0)),
        pl.BlockSpec((BN, D), lambda i: (i, 0)),
        pl.BlockSpec((BN, DEGW), lambda i: (i, 0)),
        pl.BlockSpec((BN, DEGW), lambda i: (i, 0)),
        pl.BlockSpec((BN, 1), lambda i: (i, 0)),
        pl.BlockSpec((D, D), lambda i: (0, 0)),
        pl.BlockSpec((1, D), lambda i: (0, 0)),
        pl.BlockSpec((D, C), lambda i: (0, 0)),
        pl.BlockSpec((1, C), lambda i: (0, 0)),
    ],
    out_specs=pl.BlockSpec((G, C), lambda i: (0, 0)),
    out_shape=jax.ShapeDtypeStruct((G, C), jnp.float32),
    scratch_shapes=[
        pltpu.VMEM((G, D), jnp.float32),
        pltpu.VMEM((G, D), jnp.float32),
    ],
)


def kernel(x, edge_index, graph_ids, W_emb, b_emb, W1, b1, W2, b2,
           W_out, b_out):
    src = edge_index[0].astype(jnp.int32)
    dst = edge_index[1].astype(jnp.int32)
    gid = graph_ids.astype(jnp.int32).reshape(N, 1)
    zrow = jnp.zeros((N, D), jnp.float32)
    zdeg = jnp.zeros((N, DEGW), jnp.float32)

    h0 = _embed(x, W_emb, b_emb.reshape(1, D))
    agg1, deg = _agg_deg(h0, src, dst, zrow, zdeg)
    h1 = _layer(agg1[0], agg1[1], deg[0], deg[1], W1, b1.reshape(1, D))
    agg2 = _agg_only(h1, src, dst, zrow)
    out = _head(agg2[0], agg2[1], deg[0], deg[1], gid, W2, b2.reshape(1, D),
                W_out, b_out.reshape(1, C))
    return out


# speedup vs baseline: 4.9048x; 4.9048x over previous
import functools
import jax
import jax.numpy as jnp
from jax import lax
from jax.experimental import pallas as pl
from jax.experimental.pallas import tpu as pltpu
from jax.experimental.pallas import tpu_sc as plsc

N = 10000
E = 320000
D = 128
C = 10
G = 64

NC = 2
NS = 16
NW = NC * NS
EPW = E // NW
CH = 80
NCHUNK = EPW // CH
RPT = 640
NP = NS * RPT
ZCH = 80
DEGW = 16

_mesh = plsc.VectorSubcoreMesh(core_axis_name="c", subcore_axis_name="s")


def _sc_agg_body(with_deg, *refs):
    if with_deg:
        (h_hbm, src_hbm, dst_hbm, zrow_hbm, zdeg_hbm, ones_hbm,
         out_hbm, deg_hbm, acc, dacc,
         src_v, dst_v, rows_v, ones_v, dzbuf, sem) = refs
    else:
        (h_hbm, src_hbm, dst_hbm, zrow_hbm,
         out_hbm, acc, src_v, dst_v, rows_v, sem) = refs
    cid = lax.axis_index("c")
    sid = lax.axis_index("s")
    wid = sid * NC + cid
    r0 = sid * RPT

    pltpu.sync_copy(zrow_hbm, rows_v)
    for j in range(RPT // ZCH):
        pltpu.sync_copy(rows_v, acc.at[pl.ds(r0 + j * ZCH, ZCH)])
    if with_deg:
        pltpu.sync_copy(zdeg_hbm, dzbuf)
        pltpu.sync_copy(ones_hbm, ones_v)
        for j in range(2):
            pltpu.sync_copy(dzbuf, dacc.at[pl.ds(r0 + j * (RPT // 2),
                                                 RPT // 2)])
    plsc.subcore_barrier()

    def chunk(i, carry):
        base = wid * EPW + i * CH
        pltpu.sync_copy(src_hbm.at[pl.ds(base, CH)], src_v)
        pltpu.sync_copy(dst_hbm.at[pl.ds(base, CH)], dst_v)
        pltpu.async_copy(h_hbm.at[src_v], rows_v, sem).wait()
        pltpu.sync_copy(rows_v, acc.at[dst_v], add=True)
        if with_deg:
            pltpu.sync_copy(ones_v, dacc.at[dst_v], add=True)
        return carry
    lax.fori_loop(0, NCHUNK, chunk, 0)

    plsc.subcore_barrier()
    for j in range(RPT // ZCH):
        off = r0 + j * ZCH
        pltpu.sync_copy(acc.at[pl.ds(off, ZCH)], rows_v)
        pltpu.sync_copy(rows_v, out_hbm.at[pl.ds(cid * NP + off, ZCH)])
    if with_deg:
        for j in range(2):
            off = r0 + j * (RPT // 2)
            pltpu.sync_copy(dacc.at[pl.ds(off, RPT // 2)], dzbuf)
            pltpu.sync_copy(dzbuf, deg_hbm.at[pl.ds(cid * NP + off,
                                                    RPT // 2)])


_agg_deg = functools.partial(
    pl.kernel,
    out_type=(jax.ShapeDtypeStruct((NC * NP, D), jnp.float32),
              jax.ShapeDtypeStruct((NC * NP, DEGW), jnp.float32)),
    mesh=_mesh,
    compiler_params=pltpu.CompilerParams(use_tc_tiling_on_sc=False),
    scratch_types=[
        pltpu.VMEM_SHARED((NP, D), jnp.float32),
        pltpu.VMEM_SHARED((NP, DEGW), jnp.float32),
        pltpu.VMEM((CH,), jnp.int32),
        pltpu.VMEM((CH,), jnp.int32),
        pltpu.VMEM((CH, D), jnp.float32),
        pltpu.VMEM((CH, DEGW), jnp.float32),
        pltpu.VMEM((RPT // 2, DEGW), jnp.float32),
        pltpu.SemaphoreType.DMA,
    ],
)(functools.partial(_sc_agg_body, True))

_agg_only = functools.partial(
    pl.kernel,
    out_type=jax.ShapeDtypeStruct((NC * NP, D), jnp.float32),
    mesh=_mesh,
    compiler_params=pltpu.CompilerParams(use_tc_tiling_on_sc=False),
    scratch_types=[
        pltpu.VMEM_SHARED((NP, D), jnp.float32),
        pltpu.VMEM((CH,), jnp.int32),
        pltpu.VMEM((CH,), jnp.int32),
        pltpu.VMEM((CH, D), jnp.float32),
        pltpu.SemaphoreType.DMA,
    ],
)(functools.partial(_sc_agg_body, False))


BN = 1000
NB = N // BN


def _embed_body(x_ref, w_ref, b_ref, o_ref):
    o_ref[...] = jnp.dot(x_ref[...], w_ref[...],
                         preferred_element_type=jnp.float32) + b_ref[...]


_embed = pl.pallas_call(
    _embed_body,
    grid=(NB,),
    in_specs=[
        pl.BlockSpec((BN, D), lambda i: (i, 0)),
        pl.BlockSpec((D, D), lambda i: (0, 0)),
        pl.BlockSpec((1, D), lambda i: (0, 0)),
    ],
    out_specs=pl.BlockSpec((BN, D), lambda i: (i, 0)),
    out_shape=jax.ShapeDtypeStruct((N, D), jnp.float32),
)


def _layer_body(a0_ref, a1_ref, d0_ref, d1_ref, w_ref, b_ref, o_ref):
    deg = jnp.maximum(d0_ref[:, :1] + d1_ref[:, :1], 1.0)
    h = (a0_ref[...] + a1_ref[...]) / deg
    z = jnp.dot(h, w_ref[...], preferred_element_type=jnp.float32) + b_ref[...]
    o_ref[...] = jnp.maximum(z, 0.0)


_layer = pl.pallas_call(
    _layer_body,
    grid=(NB,),
    in_specs=[
        pl.BlockSpec((BN, D), lambda i: (i, 0)),
        pl.BlockSpec((BN, D), lambda i: (i, 0)),
        pl.BlockSpec((BN, DEGW), lambda i: (i, 0)),
        pl.BlockSpec((BN, DEGW), lambda i: (i, 0)),
        pl.BlockSpec((D, D), lambda i: (0, 0)),
        pl.BlockSpec((1, D), lambda i: (0, 0)),
    ],
    out_specs=pl.BlockSpec((BN, D), lambda i: (i, 0)),
    out_shape=jax.ShapeDtypeStruct((N, D), jnp.float32),
)


def _head_body(a0_ref, a1_ref, d0_ref, d1_ref, gid_ref, w_ref, b_ref,
               wo_ref, bo_ref, o_ref, pooled_acc, cnt_acc):
    i = pl.program_id(0)

    @pl.when(i == 0)
    def _():
        pooled_acc[...] = jnp.zeros_like(pooled_acc)
        cnt_acc[...] = jnp.zeros_like(cnt_acc)

    deg = jnp.maximum(d0_ref[:, :1] + d1_ref[:, :1], 1.0)
    h = (a0_ref[...] + a1_ref[...]) / deg
    z = jnp.dot(h, w_ref[...], preferred_element_type=jnp.float32) + b_ref[...]
    h2 = jnp.maximum(z, 0.0)

    gid = gid_ref[...]
    gcol = lax.broadcasted_iota(jnp.int32, (BN, G), 1)
    onehot = (gid == gcol).astype(jnp.float32)
    dn = (((0,), (0,)), ((), ()))
    pooled_acc[...] += lax.dot_general(onehot, h2, dn,
                                       preferred_element_type=jnp.float32)
    cnt_acc[...] += lax.dot_general(onehot, jnp.ones((BN, D), jnp.float32),
                                    dn, preferred_element_type=jnp.float32)

    @pl.when(i == NB - 1)
    def _():
        pooled = pooled_acc[...] / jnp.maximum(cnt_acc[...], 1.0)
        logits = jnp.dot(pooled, wo_ref[...],
                         preferred_element_type=jnp.float32) + bo_ref[...]
        logits = jnp.where(logits >= 0, logits, 0.01 * logits)
        m = jnp.max(logits, axis=-1, keepdims=True)
        e = jnp.exp(logits - m)
        o_ref[...] = e / jnp.sum(e, axis=-1, keepdims=True)


_head = pl.pallas_call(
    _head_body,
    grid=(NB,),
    in_specs=[
        pl.BlockSpec((BN, D), lambda i: (i, 0)),
        pl.BlockSpec((BN, D), lambda i: (i, 0)),
        pl.BlockSpec((BN, DEGW), lambda i: (i, 0)),
        pl.BlockSpec((BN, DEGW), lambda i: (i, 0)),
        pl.BlockSpec((BN, 1), lambda i: (i, 0)),
        pl.BlockSpec((D, D), lambda i: (0, 0)),
        pl.BlockSpec((1, D), lambda i: (0, 0)),
        pl.BlockSpec((D, C), lambda i: (0, 0)),
        pl.BlockSpec((1, C), lambda i: (0, 0)),
    ],
    out_specs=pl.BlockSpec((G, C), lambda i: (0, 0)),
    out_shape=jax.ShapeDtypeStruct((G, C), jnp.float32),
    scratch_shapes=[
        pltpu.VMEM((G, D), jnp.float32),
        pltpu.VMEM((G, D), jnp.float32),
    ],
)


def kernel(x, edge_index, graph_ids, W_emb, b_emb, W1, b1, W2, b2,
           W_out, b_out):
    src = edge_index[0].astype(jnp.int32)
    dst = edge_index[1].astype(jnp.int32)
    gid = graph_ids.astype(jnp.int32).reshape(N, 1)
    zrow = jnp.zeros((ZCH, D), jnp.float32)
    zdeg = jnp.zeros((RPT // 2, DEGW), jnp.float32)
    ones = jnp.ones((CH, DEGW), jnp.float32)

    h0 = _embed(x, W_emb, b_emb.reshape(1, D))
    agg1, deg = _agg_deg(h0, src, dst, zrow, zdeg, ones)
    a0, a1 = agg1[:N], agg1[NP:NP + N]
    d0, d1 = deg[:N], deg[NP:NP + N]
    h1 = _layer(a0, a1, d0, d1, W1, b1.reshape(1, D))
    agg2 = _agg_only(h1, src, dst, zrow)
    out = _head(agg2[:N], agg2[NP:NP + N], d0, d1, gid,
                W2, b2.reshape(1, D), W_out, b_out.reshape(1, C))
    return out
